# baseline (speedup 1.0000x reference)
import functools
import math

import jax
import jax.numpy as jnp
from jax import lax
from jax.experimental import pallas as pl
from jax.experimental.pallas import tpu as pltpu
from jax.experimental.pallas import tpu_sc as plsc

N = 10000
E = 320000
D_IN = 128
HID = 256
HEADS = 2
NHOP = 5
NCLS = 10
NGRAPH = 64

NTILE = 16
EPT = E // NTILE
CH = 128
NCH = 160
NCH2 = NCH // 2
EPAD = NCH * CH
NPAD = 10240
RPT = NPAD // NTILE

_ROWS = 1000


def _proj_body(x_ref, W_ref, a_src_ref, a_dst_ref, h_ref, coef_ref):
    h = jnp.dot(x_ref[...], W_ref[...], preferred_element_type=jnp.float32)
    h_ref[...] = h
    hh = h.reshape(_ROWS, HEADS, HID)
    als = jnp.sum(hh * a_src_ref[...][None], axis=-1)
    ald = jnp.sum(hh * a_dst_ref[...][None], axis=-1)
    coef_ref[...] = jnp.concatenate([als, ald], axis=-1)


def _project(x, W, a_src, a_dst):
    grid = N // _ROWS
    return pl.pallas_call(
        _proj_body,
        grid=(grid,),
        in_specs=[
            pl.BlockSpec((_ROWS, D_IN), lambda i: (i, 0)),
            pl.BlockSpec((D_IN, HEADS * HID), lambda i: (0, 0)),
            pl.BlockSpec((HEADS, HID), lambda i: (0, 0)),
            pl.BlockSpec((HEADS, HID), lambda i: (0, 0)),
        ],
        out_specs=[
            pl.BlockSpec((_ROWS, HEADS * HID), lambda i: (i, 0)),
            pl.BlockSpec((_ROWS, 4), lambda i: (i, 0)),
        ],
        out_shape=[
            jax.ShapeDtypeStruct((N, HEADS * HID), jnp.float32),
            jax.ShapeDtypeStruct((N, 4), jnp.float32),
        ],
    )(x, W, a_src, a_dst)


def _final_body(u_ref, h_ref, inv_ref, lw_ref, b_ref, bias_ref, o_ref):
    acc = jnp.zeros((NGRAPH, NCLS), jnp.float32)
    for hd in range(HEADS):
        u = u_ref[:, hd * NGRAPH:(hd + 1) * NGRAPH]
        hmat = h_ref[:, hd * HID:(hd + 1) * HID]
        pooled = jax.lax.dot_general(
            u, hmat, (((0,), (0,)), ((), ())),
            preferred_element_type=jnp.float32)
        acc = acc + jnp.dot(pooled, lw_ref[...],
                            preferred_element_type=jnp.float32)
    acc = acc * inv_ref[...] / HEADS
    bias_term = jnp.dot(bias_ref[...].reshape(1, HID), lw_ref[...],
                        preferred_element_type=jnp.float32)
    o_ref[...] = acc + bias_term + b_ref[...][None, :]


def _finalize(u, h, inv_cnt, lin_w, lin_b, bias):
    return pl.pallas_call(
        _final_body,
        out_shape=jax.ShapeDtypeStruct((NGRAPH, NCLS), jnp.float32),
    )(u, h, inv_cnt, lin_w, lin_b, bias)


def _sc_body(src_hbm, dst_hbm, coef_hbm, batch_hbm, u_out_hbm,
             esrc, edst, alpha_t, as_t, ad_t, dt, bt, rb, zbuf,
             u_acc, denom_sp, sem, sem2):
    c = lax.axis_index("c")
    s = lax.axis_index("s")
    iota = lax.iota(jnp.int32, 16)

    pltpu.sync_copy(coef_hbm.at[c, 0], as_t)
    pltpu.sync_copy(coef_hbm.at[c, 1], ad_t)
    pltpu.sync_copy(batch_hbm.at[pl.ds(s * RPT, RPT)], bt)

    def zb(i, _):
        zbuf[pl.ds(i * 16, 16)] = jnp.zeros((16,), jnp.float32)
        return 0
    lax.fori_loop(0, RPT // 16, zb, 0)

    pltpu.sync_copy(zbuf.at[pl.ds(0, RPT)], denom_sp.at[pl.ds(s * RPT, RPT)])

    def zr(i, _):
        for f in range(NGRAPH // 16):
            rb[1, i, pl.ds(f * 16, 16)] = jnp.zeros((16,), jnp.float32)
        return 0
    lax.fori_loop(0, CH, zr, 0)

    def u0_block2(k, _):
        def u0_row16(rr, _):
            bv = bt[pl.ds(k * CH + rr * 16, 16)]
            for i in range(16):
                g = bv[i]
                for f in range(NGRAPH // 16):
                    rb[0, rr * 16 + i, pl.ds(f * 16, 16)] = jnp.where(
                        iota + (f * 16) == g, 1.0, 0.0)
            return 0
        lax.fori_loop(0, CH // 16, u0_row16, 0)
        row0 = s * RPT + k * CH
        pltpu.sync_copy(rb.at[0], u_out_hbm.at[c, pl.ds(row0, CH), :])
        pltpu.sync_copy(rb.at[1], u_acc.at[pl.ds(row0, CH), :])
        return 0
    lax.fori_loop(0, RPT // CH, u0_block2, 0)

    plsc.subcore_barrier()

    for half in range(2):
        pltpu.sync_copy(src_hbm.at[s, pl.ds(half * NCH2, NCH2)], esrc)
        pltpu.sync_copy(dst_hbm.at[s, pl.ds(half * NCH2, NCH2)], edst)

        def att_chunk(k, _):
            ci = half * NCH2 + k
            base = ci * CH
            for jj in range(CH // 16):
                ev_idx = base + jj * 16 + iota
                srcv = esrc[k, 0, pl.ds(jj * 16, 16)]
                dstv = edst[k, 0, pl.ds(jj * 16, 16)]
                av = plsc.load_gather(as_t, [srcv])
                dv = plsc.load_gather(ad_t, [dstv])
                e = av + dv
                e = jnp.where(e < 0.0, e * jnp.float32(0.2), e)
                ex = jnp.exp(e)
                ex = jnp.where(ev_idx < EPT, ex, 0.0)
                alpha_t[ci, 0, pl.ds(jj * 16, 16)] = ex
            pltpu.sync_copy(alpha_t.at[ci, 0], denom_sp.at[edst.at[k, 0]],
                            add=True)
            return 0
        lax.fori_loop(0, NCH2, att_chunk, 0)

    plsc.subcore_barrier()

    pltpu.sync_copy(denom_sp.at[pl.ds(0, N)], dt)
    for half in range(2):
        pltpu.sync_copy(dst_hbm.at[s, pl.ds(half * NCH2, NCH2)], edst)

        def div_chunk(k, _):
            ci = half * NCH2 + k
            for jj in range(CH // 16):
                dstv = edst[k, 0, pl.ds(jj * 16, 16)]
                dv = plsc.load_gather(dt, [dstv])
                ex = alpha_t[ci, 0, pl.ds(jj * 16, 16)]
                alpha_t[ci, 0, pl.ds(jj * 16, 16)] = ex / jnp.maximum(
                    dv, jnp.float32(1e-16))
            return 0
        lax.fori_loop(0, NCH2, div_chunk, 0)

    plsc.subcore_barrier()

    sems = [sem, sem2]
    for hop in range(NHOP):
        for half in range(2):
            pltpu.sync_copy(src_hbm.at[s, pl.ds(half * NCH2, NCH2)], esrc)
            pltpu.sync_copy(dst_hbm.at[s, pl.ds(half * NCH2, NCH2)], edst)

            pltpu.async_copy(
                u_out_hbm.at[c].at[edst.at[0, 0]], rb.at[0], sems[0])

            def hop_pair(kp, _):
                for p in range(2):
                    k = kp * 2 + p
                    ci = half * NCH2 + k
                    pltpu.make_async_copy(
                        u_out_hbm.at[c, pl.ds(0, CH), :], rb.at[p],
                        sems[p]).wait()
                    @pl.when(k + 1 < NCH2)
                    def _():
                        pltpu.async_copy(
                            u_out_hbm.at[c].at[edst.at[k + 1, 0]],
                            rb.at[1 - p], sems[1 - p])
                    def scale16(jj, _):
                        av = alpha_t[ci, 0, pl.ds(jj * 16, 16)]
                        for i in range(16):
                            r = jj * 16 + i
                            a_i = av[i]
                            for f in range(NGRAPH // 16):
                                rb[p, r, pl.ds(f * 16, 16)] = (
                                    rb[p, r, pl.ds(f * 16, 16)] * a_i)
                        return 0
                    lax.fori_loop(0, CH // 16, scale16, 0)
                    pltpu.sync_copy(rb.at[p], u_acc.at[esrc.at[k, 0]],
                                    add=True)
                return 0
            lax.fori_loop(0, NCH2 // 2, hop_pair, 0)

        plsc.subcore_barrier()

        def flush_blk(k, _):
            row0 = s * RPT + k * CH
            pltpu.sync_copy(u_acc.at[pl.ds(row0, CH), :], rb.at[0])
            pltpu.sync_copy(rb.at[0], u_out_hbm.at[c, pl.ds(row0, CH), :])
            def zb2(i, _):
                for f in range(NGRAPH // 16):
                    rb[0, i, pl.ds(f * 16, 16)] = jnp.zeros((16,), jnp.float32)
                return 0
            lax.fori_loop(0, CH, zb2, 0)
            pltpu.sync_copy(rb.at[0], u_acc.at[pl.ds(row0, CH), :])
            return 0
        lax.fori_loop(0, RPT // CH, flush_blk, 0)
        plsc.subcore_barrier()


def _sc_propagate(srcp, dstp, coefs, batchp):
    mesh = plsc.VectorSubcoreMesh(core_axis_name="c", subcore_axis_name="s")
    f = pl.kernel(
        _sc_body,
        mesh=mesh,
        compiler_params=pltpu.CompilerParams(
            needs_layout_passes=False, use_tc_tiling_on_sc=False),
        out_type=jax.ShapeDtypeStruct((HEADS, NPAD, NGRAPH), jnp.float32),
        scratch_types=[
            pltpu.VMEM((NCH2, 1, CH), jnp.int32),
            pltpu.VMEM((NCH2, 1, CH), jnp.int32),
            pltpu.VMEM((NCH, 1, CH), jnp.float32),
            pltpu.VMEM((N,), jnp.float32),
            pltpu.VMEM((N,), jnp.float32),
            pltpu.VMEM((N,), jnp.float32),
            pltpu.VMEM((RPT,), jnp.int32),
            pltpu.VMEM((2, CH, NGRAPH), jnp.float32),
            pltpu.VMEM((RPT,), jnp.float32),
            pltpu.VMEM_SHARED((NPAD, NGRAPH), jnp.float32),
            pltpu.VMEM_SHARED((NPAD,), jnp.float32),
            pltpu.SemaphoreType.DMA,
            pltpu.SemaphoreType.DMA,
        ],
    )
    return f(srcp, dstp, coefs, batchp)


def kernel(x, edge_index, batch, W, a_src, a_dst, bias, lin_w, lin_b):
    src, dst = edge_index[0], edge_index[1]
    h, coef = _project(x, W, a_src, a_dst)

    pad = jnp.zeros((NTILE, EPAD - EPT), jnp.int32)
    srcp = jnp.concatenate([src.reshape(NTILE, EPT), pad], axis=1)
    srcp = srcp.reshape(NTILE, NCH, 1, CH)
    dstp = jnp.concatenate([dst.reshape(NTILE, EPT), pad], axis=1)
    dstp = dstp.reshape(NTILE, NCH, 1, CH)
    coefs = jnp.stack([
        jnp.stack([coef[:, 0], coef[:, 2]]),
        jnp.stack([coef[:, 1], coef[:, 3]]),
    ])
    batchp = jnp.concatenate([batch, jnp.zeros((NPAD - N,), jnp.int32)])

    u5 = _sc_propagate(srcp, dstp, coefs, batchp)

    u = jnp.concatenate([u5[0, :N], u5[1, :N]], axis=1)

    u0 = (batch[:, None] == jnp.arange(NGRAPH)[None, :]).astype(jnp.float32)
    cnt = u0.sum(0)
    inv_cnt = (1.0 / jnp.maximum(cnt, 1.0))[:, None]

    return _finalize(u, h, inv_cnt, lin_w, lin_b, bias)


# speedup vs baseline: 1.2907x; 1.2907x over previous
import functools
import math

import jax
import jax.numpy as jnp
from jax import lax
from jax.experimental import pallas as pl
from jax.experimental.pallas import tpu as pltpu
from jax.experimental.pallas import tpu_sc as plsc

N = 10000
E = 320000
D_IN = 128
HID = 256
HEADS = 2
NHOP = 5
NCLS = 10
NGRAPH = 64

NTILE = 16
EPT = E // NTILE
CH = 128
NCH = 160
NCH2 = NCH // 2
EPAD = NCH * CH
NPAD = 10240
RPT = NPAD // NTILE

_ROWS = 1000


def _proj_body(x_ref, W_ref, a_src_ref, a_dst_ref, h_ref, coef_ref):
    h = jnp.dot(x_ref[...], W_ref[...], preferred_element_type=jnp.float32)
    h_ref[...] = h
    hh = h.reshape(_ROWS, HEADS, HID)
    als = jnp.sum(hh * a_src_ref[...][None], axis=-1)
    ald = jnp.sum(hh * a_dst_ref[...][None], axis=-1)
    coef_ref[...] = jnp.concatenate([als, ald], axis=-1)


def _project(x, W, a_src, a_dst):
    grid = N // _ROWS
    return pl.pallas_call(
        _proj_body,
        grid=(grid,),
        in_specs=[
            pl.BlockSpec((_ROWS, D_IN), lambda i: (i, 0)),
            pl.BlockSpec((D_IN, HEADS * HID), lambda i: (0, 0)),
            pl.BlockSpec((HEADS, HID), lambda i: (0, 0)),
            pl.BlockSpec((HEADS, HID), lambda i: (0, 0)),
        ],
        out_specs=[
            pl.BlockSpec((_ROWS, HEADS * HID), lambda i: (i, 0)),
            pl.BlockSpec((_ROWS, 4), lambda i: (i, 0)),
        ],
        out_shape=[
            jax.ShapeDtypeStruct((N, HEADS * HID), jnp.float32),
            jax.ShapeDtypeStruct((N, 4), jnp.float32),
        ],
    )(x, W, a_src, a_dst)


def _final_body(u_ref, h_ref, inv_ref, lw_ref, b_ref, bias_ref, o_ref):
    acc = jnp.zeros((NGRAPH, NCLS), jnp.float32)
    for hd in range(HEADS):
        u = u_ref[:, hd * NGRAPH:(hd + 1) * NGRAPH]
        hmat = h_ref[:, hd * HID:(hd + 1) * HID]
        pooled = jax.lax.dot_general(
            u, hmat, (((0,), (0,)), ((), ())),
            preferred_element_type=jnp.float32)
        acc = acc + jnp.dot(pooled, lw_ref[...],
                            preferred_element_type=jnp.float32)
    acc = acc * inv_ref[...] / HEADS
    bias_term = jnp.dot(bias_ref[...].reshape(1, HID), lw_ref[...],
                        preferred_element_type=jnp.float32)
    o_ref[...] = acc + bias_term + b_ref[...][None, :]


def _finalize(u, h, inv_cnt, lin_w, lin_b, bias):
    return pl.pallas_call(
        _final_body,
        out_shape=jax.ShapeDtypeStruct((NGRAPH, NCLS), jnp.float32),
    )(u, h, inv_cnt, lin_w, lin_b, bias)


def _sc_body(src_hbm, dst_hbm, coef_hbm, batch_hbm, u_out_hbm,
             esrc, edst, alpha_t, as_t, ad_t, dt, bt, rb, zbuf,
             u_acc, denom_sp, sem, sem2):
    c = lax.axis_index("c")
    s = lax.axis_index("s")
    iota = lax.iota(jnp.int32, 16)

    pltpu.sync_copy(coef_hbm.at[c, 0], as_t)
    pltpu.sync_copy(coef_hbm.at[c, 1], ad_t)
    pltpu.sync_copy(batch_hbm.at[pl.ds(s * RPT, RPT)], bt)

    def zb(i, _):
        zbuf[pl.ds(i * 16, 16)] = jnp.zeros((16,), jnp.float32)
        return 0
    lax.fori_loop(0, RPT // 16, zb, 0)

    pltpu.sync_copy(zbuf.at[pl.ds(0, RPT)], denom_sp.at[pl.ds(s * RPT, RPT)])

    def zr(i, _):
        for f in range(NGRAPH // 16):
            rb[1, i, pl.ds(f * 16, 16)] = jnp.zeros((16,), jnp.float32)
        return 0
    lax.fori_loop(0, CH, zr, 0)

    def u0_block2(k, _):
        def u0_row16(rr, _):
            bv = bt[pl.ds(k * CH + rr * 16, 16)]
            for i in range(16):
                g = bv[i]
                for f in range(NGRAPH // 16):
                    rb[0, rr * 16 + i, pl.ds(f * 16, 16)] = jnp.where(
                        iota + (f * 16) == g, 1.0, 0.0)
            return 0
        lax.fori_loop(0, CH // 16, u0_row16, 0)
        row0 = s * RPT + k * CH
        pltpu.sync_copy(rb.at[0], u_out_hbm.at[c, pl.ds(row0, CH), :])
        pltpu.sync_copy(rb.at[1], u_acc.at[pl.ds(row0, CH), :])
        return 0
    lax.fori_loop(0, RPT // CH, u0_block2, 0)

    plsc.subcore_barrier()

    for half in range(2):
        pltpu.sync_copy(src_hbm.at[s, pl.ds(half * NCH2, NCH2)], esrc)
        pltpu.sync_copy(dst_hbm.at[s, pl.ds(half * NCH2, NCH2)], edst)

        def att_chunk(k, _):
            ci = half * NCH2 + k
            base = ci * CH
            for jj in range(CH // 16):
                ev_idx = base + jj * 16 + iota
                srcv = esrc[k, 0, pl.ds(jj * 16, 16)]
                dstv = edst[k, 0, pl.ds(jj * 16, 16)]
                av = plsc.load_gather(as_t, [srcv])
                dv = plsc.load_gather(ad_t, [dstv])
                e = av + dv
                e = jnp.where(e < 0.0, e * jnp.float32(0.2), e)
                ex = jnp.exp(e)
                ex = jnp.where(ev_idx < EPT, ex, 0.0)
                alpha_t[ci, 0, pl.ds(jj * 16, 16)] = ex
            pltpu.sync_copy(alpha_t.at[ci, 0], denom_sp.at[edst.at[k, 0]],
                            add=True)
            return 0
        lax.fori_loop(0, NCH2, att_chunk, 0)

    plsc.subcore_barrier()

    pltpu.sync_copy(denom_sp.at[pl.ds(0, N)], dt)
    for half in range(2):
        pltpu.sync_copy(dst_hbm.at[s, pl.ds(half * NCH2, NCH2)], edst)

        def div_chunk(k, _):
            ci = half * NCH2 + k
            for jj in range(CH // 16):
                dstv = edst[k, 0, pl.ds(jj * 16, 16)]
                dv = plsc.load_gather(dt, [dstv])
                ex = alpha_t[ci, 0, pl.ds(jj * 16, 16)]
                alpha_t[ci, 0, pl.ds(jj * 16, 16)] = ex / jnp.maximum(
                    dv, jnp.float32(1e-16))
            return 0
        lax.fori_loop(0, NCH2, div_chunk, 0)

    plsc.subcore_barrier()

    for hop in range(NHOP):
        for half in range(2):
            pltpu.sync_copy(src_hbm.at[s, pl.ds(half * NCH2, NCH2)], esrc)
            pltpu.sync_copy(dst_hbm.at[s, pl.ds(half * NCH2, NCH2)], edst)

            def hop_chunk(k, _):
                ci = half * NCH2 + k
                pltpu.async_copy(
                    u_out_hbm.at[c].at[edst.at[k, 0]], rb.at[0], sem).wait()
                for jj in range(CH // 16):
                    av = alpha_t[ci, 0, pl.ds(jj * 16, 16)]
                    for i in range(16):
                        r = jj * 16 + i
                        a_i = av[i]
                        for f in range(NGRAPH // 16):
                            rb[0, r, pl.ds(f * 16, 16)] = (
                                rb[0, r, pl.ds(f * 16, 16)] * a_i)
                pltpu.sync_copy(rb.at[0], u_acc.at[esrc.at[k, 0]], add=True)
                return 0
            lax.fori_loop(0, NCH2, hop_chunk, 0)

        plsc.subcore_barrier()

        def flush_blk(k, _):
            row0 = s * RPT + k * CH
            pltpu.sync_copy(u_acc.at[pl.ds(row0, CH), :], rb.at[0])
            pltpu.sync_copy(rb.at[0], u_out_hbm.at[c, pl.ds(row0, CH), :])
            pltpu.sync_copy(rb.at[1], u_acc.at[pl.ds(row0, CH), :])
            return 0
        lax.fori_loop(0, RPT // CH, flush_blk, 0)
        plsc.subcore_barrier()


def _sc_propagate(srcp, dstp, coefs, batchp):
    mesh = plsc.VectorSubcoreMesh(core_axis_name="c", subcore_axis_name="s")
    f = pl.kernel(
        _sc_body,
        mesh=mesh,
        compiler_params=pltpu.CompilerParams(
            needs_layout_passes=False, use_tc_tiling_on_sc=False),
        out_type=jax.ShapeDtypeStruct((HEADS, NPAD, NGRAPH), jnp.float32),
        scratch_types=[
            pltpu.VMEM((NCH2, 1, CH), jnp.int32),
            pltpu.VMEM((NCH2, 1, CH), jnp.int32),
            pltpu.VMEM((NCH, 1, CH), jnp.float32),
            pltpu.VMEM((N,), jnp.float32),
            pltpu.VMEM((N,), jnp.float32),
            pltpu.VMEM((N,), jnp.float32),
            pltpu.VMEM((RPT,), jnp.int32),
            pltpu.VMEM((2, CH, NGRAPH), jnp.float32),
            pltpu.VMEM((RPT,), jnp.float32),
            pltpu.VMEM_SHARED((NPAD, NGRAPH), jnp.float32),
            pltpu.VMEM_SHARED((NPAD,), jnp.float32),
            pltpu.SemaphoreType.DMA,
            pltpu.SemaphoreType.DMA,
        ],
    )
    return f(srcp, dstp, coefs, batchp)


def kernel(x, edge_index, batch, W, a_src, a_dst, bias, lin_w, lin_b):
    src, dst = edge_index[0], edge_index[1]
    h, coef = _project(x, W, a_src, a_dst)

    pad = jnp.zeros((NTILE, EPAD - EPT), jnp.int32)
    srcp = jnp.concatenate([src.reshape(NTILE, EPT), pad], axis=1)
    srcp = srcp.reshape(NTILE, NCH, 1, CH)
    dstp = jnp.concatenate([dst.reshape(NTILE, EPT), pad], axis=1)
    dstp = dstp.reshape(NTILE, NCH, 1, CH)
    coefs = jnp.stack([
        jnp.stack([coef[:, 0], coef[:, 2]]),
        jnp.stack([coef[:, 1], coef[:, 3]]),
    ])
    batchp = jnp.concatenate([batch, jnp.zeros((NPAD - N,), jnp.int32)])

    u5 = _sc_propagate(srcp, dstp, coefs, batchp)

    u = jnp.concatenate([u5[0, :N], u5[1, :N]], axis=1)

    u0 = (batch[:, None] == jnp.arange(NGRAPH)[None, :]).astype(jnp.float32)
    cnt = u0.sum(0)
    inv_cnt = (1.0 / jnp.maximum(cnt, 1.0))[:, None]

    return _finalize(u, h, inv_cnt, lin_w, lin_b, bias)


# speedup vs baseline: 1.6587x; 1.2851x over previous
import functools
import math

import jax
import jax.numpy as jnp
from jax import lax
from jax.experimental import pallas as pl
from jax.experimental.pallas import tpu as pltpu
from jax.experimental.pallas import tpu_sc as plsc

N = 10000
E = 320000
D_IN = 128
HID = 256
HEADS = 2
NHOP = 5
NCLS = 10
NGRAPH = 64

NTILE = 16
EPT = E // NTILE
CH = 128
NCH = 160
NCH2 = NCH // 2
EPAD = NCH * CH
NPAD = 10240
RPT = NPAD // NTILE

_ROWS = 1000


def _proj_body(x_ref, W_ref, a_src_ref, a_dst_ref, h_ref, coef_ref):
    h = jnp.dot(x_ref[...], W_ref[...], preferred_element_type=jnp.float32)
    h_ref[...] = h
    hh = h.reshape(_ROWS, HEADS, HID)
    als = jnp.sum(hh * a_src_ref[...][None], axis=-1)
    ald = jnp.sum(hh * a_dst_ref[...][None], axis=-1)
    coef_ref[...] = jnp.concatenate([als, ald], axis=-1)


def _project(x, W, a_src, a_dst):
    grid = N // _ROWS
    return pl.pallas_call(
        _proj_body,
        grid=(grid,),
        in_specs=[
            pl.BlockSpec((_ROWS, D_IN), lambda i: (i, 0)),
            pl.BlockSpec((D_IN, HEADS * HID), lambda i: (0, 0)),
            pl.BlockSpec((HEADS, HID), lambda i: (0, 0)),
            pl.BlockSpec((HEADS, HID), lambda i: (0, 0)),
        ],
        out_specs=[
            pl.BlockSpec((_ROWS, HEADS * HID), lambda i: (i, 0)),
            pl.BlockSpec((_ROWS, 4), lambda i: (i, 0)),
        ],
        out_shape=[
            jax.ShapeDtypeStruct((N, HEADS * HID), jnp.float32),
            jax.ShapeDtypeStruct((N, 4), jnp.float32),
        ],
    )(x, W, a_src, a_dst)


def _final_body(u_ref, h_ref, inv_ref, lw_ref, b_ref, bias_ref, o_ref):
    acc = jnp.zeros((NGRAPH, NCLS), jnp.float32)
    for hd in range(HEADS):
        u = u_ref[:, hd * NGRAPH:(hd + 1) * NGRAPH]
        hmat = h_ref[:, hd * HID:(hd + 1) * HID]
        pooled = jax.lax.dot_general(
            u, hmat, (((0,), (0,)), ((), ())),
            preferred_element_type=jnp.float32)
        acc = acc + jnp.dot(pooled, lw_ref[...],
                            preferred_element_type=jnp.float32)
    acc = acc * inv_ref[...] / HEADS
    bias_term = jnp.dot(bias_ref[...].reshape(1, HID), lw_ref[...],
                        preferred_element_type=jnp.float32)
    o_ref[...] = acc + bias_term + b_ref[...][None, :]


def _finalize(u, h, inv_cnt, lin_w, lin_b, bias):
    return pl.pallas_call(
        _final_body,
        out_shape=jax.ShapeDtypeStruct((NGRAPH, NCLS), jnp.float32),
    )(u, h, inv_cnt, lin_w, lin_b, bias)


def _sc_body(src_hbm, dst_hbm, coef_hbm, batch_hbm, u_out_hbm,
             esrc, edst, alpha_t, as_t, ad_t, dv, bt, rb, zblk, zbuf,
             u_acc, denom_sp, sem, sem2):
    c = lax.axis_index("c")
    s = lax.axis_index("s")
    iota = lax.iota(jnp.int32, 16)
    sems = [sem, sem2]

    pltpu.sync_copy(coef_hbm.at[c, 0], as_t)
    pltpu.sync_copy(coef_hbm.at[c, 1], ad_t)
    pltpu.sync_copy(batch_hbm.at[pl.ds(s * RPT, RPT)], bt)

    def zb(i, _):
        zbuf[pl.ds(i * 16, 16)] = jnp.zeros((16,), jnp.float32)
        return 0
    lax.fori_loop(0, RPT // 16, zb, 0)

    pltpu.sync_copy(zbuf.at[pl.ds(0, RPT)], denom_sp.at[pl.ds(s * RPT, RPT)])

    def zr(i, _):
        for f in range(NGRAPH // 16):
            zblk[i, pl.ds(f * 16, 16)] = jnp.zeros((16,), jnp.float32)
        return 0
    lax.fori_loop(0, CH // 2, zr, 0)

    def u0_block2(k, _):
        def u0_row16(rr, _):
            bv = bt[pl.ds(k * CH + rr * 16, 16)]
            for i in range(16):
                g = bv[i]
                for f in range(NGRAPH // 16):
                    rb[0, rr * 16 + i, pl.ds(f * 16, 16)] = jnp.where(
                        iota + (f * 16) == g, 1.0, 0.0)
            return 0
        lax.fori_loop(0, CH // 16, u0_row16, 0)
        row0 = s * RPT + k * CH
        pltpu.sync_copy(rb.at[0], u_out_hbm.at[c, pl.ds(row0, CH), :])
        pltpu.sync_copy(zblk, u_acc.at[pl.ds(row0, CH // 2), :])
        pltpu.sync_copy(zblk, u_acc.at[pl.ds(row0 + CH // 2, CH // 2), :])
        return 0
    lax.fori_loop(0, RPT // CH, u0_block2, 0)

    plsc.subcore_barrier()

    for half in range(2):
        pltpu.sync_copy(src_hbm.at[s, pl.ds(half * NCH2, NCH2)], esrc)
        pltpu.sync_copy(dst_hbm.at[s, pl.ds(half * NCH2, NCH2)], edst)

        def att_chunk(k, _):
            ci = half * NCH2 + k
            base = ci * CH
            for jj in range(CH // 16):
                ev_idx = base + jj * 16 + iota
                srcv = esrc[k, 0, pl.ds(jj * 16, 16)]
                dstv = edst[k, 0, pl.ds(jj * 16, 16)]
                av = plsc.load_gather(as_t, [srcv])
                dv = plsc.load_gather(ad_t, [dstv])
                e = av + dv
                e = jnp.where(e < 0.0, e * jnp.float32(0.2), e)
                ex = jnp.exp(e)
                ex = jnp.where(ev_idx < EPT, ex, 0.0)
                alpha_t[ci, 0, pl.ds(jj * 16, 16)] = ex
            pltpu.sync_copy(alpha_t.at[ci, 0], denom_sp.at[edst.at[k, 0]],
                            add=True)
            return 0
        lax.fori_loop(0, NCH2, att_chunk, 0)

    plsc.subcore_barrier()

    for half in range(2):
        pltpu.sync_copy(dst_hbm.at[s, pl.ds(half * NCH2, NCH2)], edst)

        def div_chunk(k, _):
            ci = half * NCH2 + k
            pltpu.async_copy(denom_sp.at[edst.at[k, 0]], dv, sem).wait()
            for jj in range(CH // 16):
                dv16 = dv[pl.ds(jj * 16, 16)]
                ex = alpha_t[ci, 0, pl.ds(jj * 16, 16)]
                alpha_t[ci, 0, pl.ds(jj * 16, 16)] = ex / jnp.maximum(
                    dv16, jnp.float32(1e-16))
            return 0
        lax.fori_loop(0, NCH2, div_chunk, 0)

    plsc.subcore_barrier()

    def hop_body(hop, _):
        def half_body(half, _):
            off = half * NCH2
            pltpu.sync_copy(src_hbm.at[s, pl.ds(off, NCH2)], esrc)
            pltpu.sync_copy(dst_hbm.at[s, pl.ds(off, NCH2)], edst)

            pltpu.async_copy(
                u_out_hbm.at[c].at[edst.at[0, 0]], rb.at[0], sems[0])

            def hop_pair(kp, _):
                for p in range(2):
                    k = kp * 2 + p
                    ci = off + k
                    pltpu.make_async_copy(
                        u_out_hbm.at[c, pl.ds(0, CH), :], rb.at[p],
                        sems[p]).wait()
                    @pl.when(k + 1 < NCH2)
                    def _():
                        pltpu.async_copy(
                            u_out_hbm.at[c].at[edst.at[k + 1, 0]],
                            rb.at[1 - p], sems[1 - p])
                    for jj in range(CH // 16):
                        av = alpha_t[ci, 0, pl.ds(jj * 16, 16)]
                        for i in range(16):
                            r = jj * 16 + i
                            a_i = av[i]
                            for f in range(NGRAPH // 16):
                                rb[p, r, pl.ds(f * 16, 16)] = (
                                    rb[p, r, pl.ds(f * 16, 16)] * a_i)
                    pltpu.sync_copy(rb.at[p], u_acc.at[esrc.at[k, 0]],
                                    add=True)
                return 0
            lax.fori_loop(0, NCH2 // 2, hop_pair, 0)
            return 0
        lax.fori_loop(0, 2, half_body, 0)

        plsc.subcore_barrier()

        def flush_blk(k, _):
            row0 = s * RPT + k * CH
            pltpu.sync_copy(u_acc.at[pl.ds(row0, CH), :], rb.at[0])
            pltpu.sync_copy(rb.at[0], u_out_hbm.at[c, pl.ds(row0, CH), :])
            pltpu.sync_copy(zblk, u_acc.at[pl.ds(row0, CH // 2), :])
            pltpu.sync_copy(zblk, u_acc.at[pl.ds(row0 + CH // 2, CH // 2), :])
            return 0
        lax.fori_loop(0, RPT // CH, flush_blk, 0)
        plsc.subcore_barrier()
        return 0
    lax.fori_loop(0, NHOP, hop_body, 0)


def _sc_propagate(srcp, dstp, coefs, batchp):
    mesh = plsc.VectorSubcoreMesh(core_axis_name="c", subcore_axis_name="s")
    f = pl.kernel(
        _sc_body,
        mesh=mesh,
        compiler_params=pltpu.CompilerParams(
            needs_layout_passes=False, use_tc_tiling_on_sc=False),
        out_type=jax.ShapeDtypeStruct((HEADS, NPAD, NGRAPH), jnp.float32),
        scratch_types=[
            pltpu.VMEM((NCH2, 1, CH), jnp.int32),
            pltpu.VMEM((NCH2, 1, CH), jnp.int32),
            pltpu.VMEM((NCH, 1, CH), jnp.float32),
            pltpu.VMEM((N,), jnp.float32),
            pltpu.VMEM((N,), jnp.float32),
            pltpu.VMEM((CH,), jnp.float32),
            pltpu.VMEM((RPT,), jnp.int32),
            pltpu.VMEM((2, CH, NGRAPH), jnp.float32),
            pltpu.VMEM((CH // 2, NGRAPH), jnp.float32),
            pltpu.VMEM((RPT,), jnp.float32),
            pltpu.VMEM_SHARED((NPAD, NGRAPH), jnp.float32),
            pltpu.VMEM_SHARED((NPAD,), jnp.float32),
            pltpu.SemaphoreType.DMA,
            pltpu.SemaphoreType.DMA,
        ],
    )
    return f(srcp, dstp, coefs, batchp)


def kernel(x, edge_index, batch, W, a_src, a_dst, bias, lin_w, lin_b):
    src, dst = edge_index[0], edge_index[1]
    h, coef = _project(x, W, a_src, a_dst)

    pad = jnp.zeros((NTILE, EPAD - EPT), jnp.int32)
    srcp = jnp.concatenate([src.reshape(NTILE, EPT), pad], axis=1)
    srcp = srcp.reshape(NTILE, NCH, 1, CH)
    dstp = jnp.concatenate([dst.reshape(NTILE, EPT), pad], axis=1)
    dstp = dstp.reshape(NTILE, NCH, 1, CH)
    coefs = jnp.stack([
        jnp.stack([coef[:, 0], coef[:, 2]]),
        jnp.stack([coef[:, 1], coef[:, 3]]),
    ])
    batchp = jnp.concatenate([batch, jnp.zeros((NPAD - N,), jnp.int32)])

    u5 = _sc_propagate(srcp, dstp, coefs, batchp)

    u = jnp.concatenate([u5[0, :N], u5[1, :N]], axis=1)

    u0 = (batch[:, None] == jnp.arange(NGRAPH)[None, :]).astype(jnp.float32)
    cnt = u0.sum(0)
    inv_cnt = (1.0 / jnp.maximum(cnt, 1.0))[:, None]

    return _finalize(u, h, inv_cnt, lin_w, lin_b, bias)


# speedup vs baseline: 1.6700x; 1.0068x over previous
import functools
import math

import jax
import jax.numpy as jnp
from jax import lax
from jax.experimental import pallas as pl
from jax.experimental.pallas import tpu as pltpu
from jax.experimental.pallas import tpu_sc as plsc

N = 10000
E = 320000
D_IN = 128
HID = 256
HEADS = 2
NHOP = 5
NCLS = 10
NGRAPH = 64

NTILE = 16
EPT = E // NTILE
CH = 128
NCH = 160
NCH2 = NCH // 2
EPAD = NCH * CH
NPAD = 10240
RPT = NPAD // NTILE

_ROWS = 1000


def _proj_body(x_ref, W_ref, a_src_ref, a_dst_ref, h_ref, coef_ref):
    h = jnp.dot(x_ref[...], W_ref[...], preferred_element_type=jnp.float32)
    h_ref[...] = h
    hh = h.reshape(_ROWS, HEADS, HID)
    als = jnp.sum(hh * a_src_ref[...][None], axis=-1)
    ald = jnp.sum(hh * a_dst_ref[...][None], axis=-1)
    coef_ref[...] = jnp.concatenate([als, ald], axis=-1)


def _project(x, W, a_src, a_dst):
    grid = N // _ROWS
    return pl.pallas_call(
        _proj_body,
        grid=(grid,),
        in_specs=[
            pl.BlockSpec((_ROWS, D_IN), lambda i: (i, 0)),
            pl.BlockSpec((D_IN, HEADS * HID), lambda i: (0, 0)),
            pl.BlockSpec((HEADS, HID), lambda i: (0, 0)),
            pl.BlockSpec((HEADS, HID), lambda i: (0, 0)),
        ],
        out_specs=[
            pl.BlockSpec((_ROWS, HEADS * HID), lambda i: (i, 0)),
            pl.BlockSpec((_ROWS, 4), lambda i: (i, 0)),
        ],
        out_shape=[
            jax.ShapeDtypeStruct((N, HEADS * HID), jnp.float32),
            jax.ShapeDtypeStruct((N, 4), jnp.float32),
        ],
    )(x, W, a_src, a_dst)


def _final_body(u_ref, h_ref, inv_ref, lw_ref, b_ref, bias_ref, o_ref):
    acc = jnp.zeros((NGRAPH, NCLS), jnp.float32)
    for hd in range(HEADS):
        u = u_ref[:, hd * NGRAPH:(hd + 1) * NGRAPH]
        hmat = h_ref[:, hd * HID:(hd + 1) * HID]
        pooled = jax.lax.dot_general(
            u, hmat, (((0,), (0,)), ((), ())),
            preferred_element_type=jnp.float32)
        acc = acc + jnp.dot(pooled, lw_ref[...],
                            preferred_element_type=jnp.float32)
    acc = acc * inv_ref[...] / HEADS
    bias_term = jnp.dot(bias_ref[...].reshape(1, HID), lw_ref[...],
                        preferred_element_type=jnp.float32)
    o_ref[...] = acc + bias_term + b_ref[...][None, :]


def _finalize(u, h, inv_cnt, lin_w, lin_b, bias):
    return pl.pallas_call(
        _final_body,
        out_shape=jax.ShapeDtypeStruct((NGRAPH, NCLS), jnp.float32),
    )(u, h, inv_cnt, lin_w, lin_b, bias)


def _sc_body(src_hbm, dst_hbm, coef_hbm, batch_hbm, u_out_hbm,
             esrc, edst, alpha_t, as_t, ad_t, dv, bt, rb, zblk, zbuf,
             u_acc, denom_sp, sem, sem2, sem3, sem4):
    c = lax.axis_index("c")
    s = lax.axis_index("s")
    iota = lax.iota(jnp.int32, 16)
    sems = [sem, sem2]
    sems_sc = [sem3, sem4]

    pltpu.sync_copy(coef_hbm.at[c, 0], as_t)
    pltpu.sync_copy(coef_hbm.at[c, 1], ad_t)
    pltpu.sync_copy(batch_hbm.at[pl.ds(s * RPT, RPT)], bt)

    def zb(i, _):
        zbuf[pl.ds(i * 16, 16)] = jnp.zeros((16,), jnp.float32)
        return 0
    lax.fori_loop(0, RPT // 16, zb, 0)

    pltpu.sync_copy(zbuf.at[pl.ds(0, RPT)], denom_sp.at[pl.ds(s * RPT, RPT)])

    def zr(i, _):
        for f in range(NGRAPH // 16):
            zblk[i, pl.ds(f * 16, 16)] = jnp.zeros((16,), jnp.float32)
        return 0
    lax.fori_loop(0, CH // 2, zr, 0)

    def u0_block2(k, _):
        def u0_row16(rr, _):
            bv = bt[pl.ds(k * CH + rr * 16, 16)]
            for i in range(16):
                g = bv[i]
                for f in range(NGRAPH // 16):
                    rb[0, rr * 16 + i, pl.ds(f * 16, 16)] = jnp.where(
                        iota + (f * 16) == g, 1.0, 0.0)
            return 0
        lax.fori_loop(0, CH // 16, u0_row16, 0)
        row0 = s * RPT + k * CH
        pltpu.sync_copy(rb.at[0], u_out_hbm.at[c, pl.ds(row0, CH), :])
        pltpu.sync_copy(zblk, u_acc.at[pl.ds(row0, CH // 2), :])
        pltpu.sync_copy(zblk, u_acc.at[pl.ds(row0 + CH // 2, CH // 2), :])
        return 0
    lax.fori_loop(0, RPT // CH, u0_block2, 0)

    plsc.subcore_barrier()

    for half in range(2):
        pltpu.sync_copy(src_hbm.at[s, pl.ds(half * NCH2, NCH2)], esrc)
        pltpu.sync_copy(dst_hbm.at[s, pl.ds(half * NCH2, NCH2)], edst)

        def att_chunk(k, _):
            ci = half * NCH2 + k
            base = ci * CH
            for jj in range(CH // 16):
                ev_idx = base + jj * 16 + iota
                srcv = esrc[k, 0, pl.ds(jj * 16, 16)]
                dstv = edst[k, 0, pl.ds(jj * 16, 16)]
                av = plsc.load_gather(as_t, [srcv])
                dv = plsc.load_gather(ad_t, [dstv])
                e = av + dv
                e = jnp.where(e < 0.0, e * jnp.float32(0.2), e)
                ex = jnp.exp(e)
                ex = jnp.where(ev_idx < EPT, ex, 0.0)
                alpha_t[ci, 0, pl.ds(jj * 16, 16)] = ex
            pltpu.async_copy(alpha_t.at[ci, 0], denom_sp.at[edst.at[k, 0]],
                             sem3, add=True)
            return 0
        lax.fori_loop(0, NCH2, att_chunk, 0)

        def att_drain(k, _):
            pltpu.make_async_copy(alpha_t.at[0, 0], denom_sp.at[pl.ds(0, CH)],
                                  sem3).wait()
            return 0
        lax.fori_loop(0, NCH2, att_drain, 0)

    plsc.subcore_barrier()

    for half in range(2):
        pltpu.sync_copy(dst_hbm.at[s, pl.ds(half * NCH2, NCH2)], edst)

        def div_chunk(k, _):
            ci = half * NCH2 + k
            pltpu.async_copy(denom_sp.at[edst.at[k, 0]], dv, sem).wait()
            for jj in range(CH // 16):
                dv16 = dv[pl.ds(jj * 16, 16)]
                ex = alpha_t[ci, 0, pl.ds(jj * 16, 16)]
                alpha_t[ci, 0, pl.ds(jj * 16, 16)] = ex / jnp.maximum(
                    dv16, jnp.float32(1e-16))
            return 0
        lax.fori_loop(0, NCH2, div_chunk, 0)

    plsc.subcore_barrier()

    def hop_body(hop, _):
        def half_body(half, _):
            off = half * NCH2
            pltpu.sync_copy(src_hbm.at[s, pl.ds(off, NCH2)], esrc)
            pltpu.sync_copy(dst_hbm.at[s, pl.ds(off, NCH2)], edst)

            pltpu.async_copy(
                u_out_hbm.at[c].at[edst.at[0, 0]], rb.at[0], sems[0])

            def hop_pair(kp, _):
                for p in range(2):
                    k = kp * 2 + p
                    ci = off + k
                    pltpu.make_async_copy(
                        u_out_hbm.at[c, pl.ds(0, CH), :], rb.at[p],
                        sems[p]).wait()
                    @pl.when(k >= 1)
                    def _():
                        pltpu.make_async_copy(
                            rb.at[1 - p], u_acc.at[pl.ds(0, CH), :],
                            sems_sc[1 - p]).wait()
                    @pl.when(k + 1 < NCH2)
                    def _():
                        pltpu.async_copy(
                            u_out_hbm.at[c].at[edst.at[k + 1, 0]],
                            rb.at[1 - p], sems[1 - p])
                    for jj in range(CH // 16):
                        av = alpha_t[ci, 0, pl.ds(jj * 16, 16)]
                        for i in range(16):
                            r = jj * 16 + i
                            a_i = av[i]
                            for f in range(NGRAPH // 16):
                                rb[p, r, pl.ds(f * 16, 16)] = (
                                    rb[p, r, pl.ds(f * 16, 16)] * a_i)
                    pltpu.async_copy(rb.at[p], u_acc.at[esrc.at[k, 0]],
                                     sems_sc[p], add=True)
                return 0
            lax.fori_loop(0, NCH2 // 2, hop_pair, 0)
            pltpu.make_async_copy(
                rb.at[(NCH2 - 1) % 2], u_acc.at[pl.ds(0, CH), :],
                sems_sc[(NCH2 - 1) % 2]).wait()
            return 0
        lax.fori_loop(0, 2, half_body, 0)

        plsc.subcore_barrier()

        def flush_blk(k, _):
            row0 = s * RPT + k * CH
            pltpu.sync_copy(u_acc.at[pl.ds(row0, CH), :], rb.at[0])
            pltpu.sync_copy(rb.at[0], u_out_hbm.at[c, pl.ds(row0, CH), :])
            pltpu.sync_copy(zblk, u_acc.at[pl.ds(row0, CH // 2), :])
            pltpu.sync_copy(zblk, u_acc.at[pl.ds(row0 + CH // 2, CH // 2), :])
            return 0
        lax.fori_loop(0, RPT // CH, flush_blk, 0)
        plsc.subcore_barrier()
        return 0
    lax.fori_loop(0, NHOP, hop_body, 0)


def _sc_propagate(srcp, dstp, coefs, batchp):
    mesh = plsc.VectorSubcoreMesh(core_axis_name="c", subcore_axis_name="s")
    f = pl.kernel(
        _sc_body,
        mesh=mesh,
        compiler_params=pltpu.CompilerParams(
            needs_layout_passes=False, use_tc_tiling_on_sc=False),
        out_type=jax.ShapeDtypeStruct((HEADS, NPAD, NGRAPH), jnp.float32),
        scratch_types=[
            pltpu.VMEM((NCH2, 1, CH), jnp.int32),
            pltpu.VMEM((NCH2, 1, CH), jnp.int32),
            pltpu.VMEM((NCH, 1, CH), jnp.float32),
            pltpu.VMEM((N,), jnp.float32),
            pltpu.VMEM((N,), jnp.float32),
            pltpu.VMEM((CH,), jnp.float32),
            pltpu.VMEM((RPT,), jnp.int32),
            pltpu.VMEM((2, CH, NGRAPH), jnp.float32),
            pltpu.VMEM((CH // 2, NGRAPH), jnp.float32),
            pltpu.VMEM((RPT,), jnp.float32),
            pltpu.VMEM_SHARED((NPAD, NGRAPH), jnp.float32),
            pltpu.VMEM_SHARED((NPAD,), jnp.float32),
            pltpu.SemaphoreType.DMA,
            pltpu.SemaphoreType.DMA,
            pltpu.SemaphoreType.DMA,
            pltpu.SemaphoreType.DMA,
        ],
    )
    return f(srcp, dstp, coefs, batchp)


def kernel(x, edge_index, batch, W, a_src, a_dst, bias, lin_w, lin_b):
    src, dst = edge_index[0], edge_index[1]
    h, coef = _project(x, W, a_src, a_dst)

    pad = jnp.zeros((NTILE, EPAD - EPT), jnp.int32)
    srcp = jnp.concatenate([src.reshape(NTILE, EPT), pad], axis=1)
    srcp = srcp.reshape(NTILE, NCH, 1, CH)
    dstp = jnp.concatenate([dst.reshape(NTILE, EPT), pad], axis=1)
    dstp = dstp.reshape(NTILE, NCH, 1, CH)
    coefs = jnp.stack([
        jnp.stack([coef[:, 0], coef[:, 2]]),
        jnp.stack([coef[:, 1], coef[:, 3]]),
    ])
    batchp = jnp.concatenate([batch, jnp.zeros((NPAD - N,), jnp.int32)])

    u5 = _sc_propagate(srcp, dstp, coefs, batchp)

    u = jnp.concatenate([u5[0, :N], u5[1, :N]], axis=1)

    u0 = (batch[:, None] == jnp.arange(NGRAPH)[None, :]).astype(jnp.float32)
    cnt = u0.sum(0)
    inv_cnt = (1.0 / jnp.maximum(cnt, 1.0))[:, None]

    return _finalize(u, h, inv_cnt, lin_w, lin_b, bias)


# speedup vs baseline: 1.6729x; 1.0018x over previous
import functools
import math

import jax
import jax.numpy as jnp
from jax import lax
from jax.experimental import pallas as pl
from jax.experimental.pallas import tpu as pltpu
from jax.experimental.pallas import tpu_sc as plsc

N = 10000
E = 320000
D_IN = 128
HID = 256
HEADS = 2
NHOP = 5
NCLS = 10
NGRAPH = 64

NTILE = 16
EPT = E // NTILE
CH = 128
NCH = 160
NCH2 = NCH // 2
NCH4 = NCH // 4
EPAD = NCH * CH
NPAD = 10240
RPT = NPAD // NTILE
NDR = NPAD // 16

_ROWS = 1000


def _proj_body(x_ref, W_ref, a_src_ref, a_dst_ref, h_ref, coef_ref):
    h = jnp.dot(x_ref[...], W_ref[...], preferred_element_type=jnp.float32)
    h_ref[...] = h
    hh = h.reshape(_ROWS, HEADS, HID)
    als = jnp.sum(hh * a_src_ref[...][None], axis=-1)
    ald = jnp.sum(hh * a_dst_ref[...][None], axis=-1)
    coef_ref[...] = jnp.concatenate([als, ald], axis=-1)


def _project(x, W, a_src, a_dst):
    grid = N // _ROWS
    return pl.pallas_call(
        _proj_body,
        grid=(grid,),
        in_specs=[
            pl.BlockSpec((_ROWS, D_IN), lambda i: (i, 0)),
            pl.BlockSpec((D_IN, HEADS * HID), lambda i: (0, 0)),
            pl.BlockSpec((HEADS, HID), lambda i: (0, 0)),
            pl.BlockSpec((HEADS, HID), lambda i: (0, 0)),
        ],
        out_specs=[
            pl.BlockSpec((_ROWS, HEADS * HID), lambda i: (i, 0)),
            pl.BlockSpec((_ROWS, 4), lambda i: (i, 0)),
        ],
        out_shape=[
            jax.ShapeDtypeStruct((N, HEADS * HID), jnp.float32),
            jax.ShapeDtypeStruct((N, 4), jnp.float32),
        ],
    )(x, W, a_src, a_dst)


def _final_body(u_ref, h_ref, inv_ref, lw_ref, b_ref, bias_ref, o_ref):
    acc = jnp.zeros((NGRAPH, NCLS), jnp.float32)
    for hd in range(HEADS):
        u = u_ref[:, hd * NGRAPH:(hd + 1) * NGRAPH]
        hmat = h_ref[:, hd * HID:(hd + 1) * HID]
        pooled = jax.lax.dot_general(
            u, hmat, (((0,), (0,)), ((), ())),
            preferred_element_type=jnp.float32)
        acc = acc + jnp.dot(pooled, lw_ref[...],
                            preferred_element_type=jnp.float32)
    acc = acc * inv_ref[...] / HEADS
    bias_term = jnp.dot(bias_ref[...].reshape(1, HID), lw_ref[...],
                        preferred_element_type=jnp.float32)
    o_ref[...] = acc + bias_term + b_ref[...][None, :]


def _finalize(u, h, inv_cnt, lin_w, lin_b, bias):
    return pl.pallas_call(
        _final_body,
        out_shape=jax.ShapeDtypeStruct((NGRAPH, NCLS), jnp.float32),
    )(u, h, inv_cnt, lin_w, lin_b, bias)


def _sc_body(src_hbm, dst_hbm, coef_hbm, batch_hbm, u_out_hbm,
             esrc, edst, alpha_t, as_t, ad_t, dl, ri, bt, rb, zblk,
             u_acc, denom_sp, sem, sem2, sem3, sem4):
    c = lax.axis_index("c")
    s = lax.axis_index("s")
    iota = lax.iota(jnp.int32, 16)
    sems = [sem, sem2]
    sems_sc = [sem3, sem4]

    pltpu.sync_copy(coef_hbm.at[c, 0], as_t)
    pltpu.sync_copy(coef_hbm.at[c, 1], ad_t)
    pltpu.sync_copy(batch_hbm.at[pl.ds(s * RPT, RPT)], bt)

    def zdl(i, _):
        dl[i, :] = jnp.zeros((16,), jnp.float32)
        return 0
    lax.fori_loop(0, NDR, zdl, 0)
    def zri(i, _):
        ri[pl.ds(i * 16, 16)] = iota + i * 16
        return 0
    lax.fori_loop(0, NDR // 16, zri, 0)

    pltpu.sync_copy(dl.at[pl.ds(s * (NDR // NTILE), NDR // NTILE), :],
                    denom_sp.at[pl.ds(s * (NDR // NTILE), NDR // NTILE), :])

    def zr(i, _):
        for f in range(NGRAPH // 16):
            zblk[i, pl.ds(f * 16, 16)] = jnp.zeros((16,), jnp.float32)
        return 0
    lax.fori_loop(0, CH // 2, zr, 0)

    def u0_block2(k, _):
        def u0_row16(rr, _):
            bv = bt[pl.ds(k * CH + rr * 16, 16)]
            for i in range(16):
                g = bv[i]
                for f in range(NGRAPH // 16):
                    rb[0, rr * 16 + i, pl.ds(f * 16, 16)] = jnp.where(
                        iota + (f * 16) == g, 1.0, 0.0)
            return 0
        lax.fori_loop(0, CH // 16, u0_row16, 0)
        row0 = s * RPT + k * CH
        pltpu.sync_copy(rb.at[0], u_out_hbm.at[c, pl.ds(row0, CH), :])
        pltpu.sync_copy(zblk, u_acc.at[pl.ds(row0, CH // 2), :])
        pltpu.sync_copy(zblk, u_acc.at[pl.ds(row0 + CH // 2, CH // 2), :])
        return 0
    lax.fori_loop(0, RPT // CH, u0_block2, 0)

    plsc.subcore_barrier()

    for q in range(4):
        pltpu.sync_copy(src_hbm.at[s, pl.ds(q * NCH4, NCH4)], esrc)
        pltpu.sync_copy(dst_hbm.at[s, pl.ds(q * NCH4, NCH4)], edst)

        def att_chunk(k, _):
            ci = q * NCH4 + k
            base = ci * CH
            for jj in range(CH // 16):
                ev_idx = base + jj * 16 + iota
                srcv = esrc[k, 0, pl.ds(jj * 16, 16)]
                dstv = edst[k, 0, pl.ds(jj * 16, 16)]
                av = plsc.load_gather(as_t, [srcv])
                dv16 = plsc.load_gather(ad_t, [dstv])
                e = av + dv16
                e = jnp.where(e < 0.0, e * jnp.float32(0.2), e)
                ex = jnp.exp(e)
                ex = jnp.where(ev_idx < EPT, ex, 0.0)
                alpha_t[ci, 0, pl.ds(jj * 16, 16)] = ex
                plsc.addupdate_scatter(
                    dl, [dstv >> 4, dstv & 15], ex)
            return 0
        lax.fori_loop(0, NCH4, att_chunk, 0)

    pltpu.sync_copy(dl, denom_sp.at[ri], add=True)
    plsc.subcore_barrier()
    pltpu.sync_copy(denom_sp, dl)

    for q in range(4):
        pltpu.sync_copy(dst_hbm.at[s, pl.ds(q * NCH4, NCH4)], edst)

        def div_chunk(k, _):
            ci = q * NCH4 + k
            for jj in range(CH // 16):
                dstv = edst[k, 0, pl.ds(jj * 16, 16)]
                dv16 = plsc.load_gather(dl, [dstv >> 4, dstv & 15])
                ex = alpha_t[ci, 0, pl.ds(jj * 16, 16)]
                alpha_t[ci, 0, pl.ds(jj * 16, 16)] = ex / jnp.maximum(
                    dv16, jnp.float32(1e-16))
            return 0
        lax.fori_loop(0, NCH4, div_chunk, 0)

    plsc.subcore_barrier()

    def hop_body(hop, _):
        def quarter_body(q, _):
            off = q * NCH4
            pltpu.sync_copy(src_hbm.at[s, pl.ds(off, NCH4)], esrc)
            pltpu.sync_copy(dst_hbm.at[s, pl.ds(off, NCH4)], edst)

            pltpu.async_copy(
                u_out_hbm.at[c].at[edst.at[0, 0]], rb.at[0], sems[0])

            def hop_pair(kp, _):
                for p in range(2):
                    k = kp * 2 + p
                    ci = off + k
                    pltpu.make_async_copy(
                        u_out_hbm.at[c, pl.ds(0, CH), :], rb.at[p],
                        sems[p]).wait()
                    @pl.when(k >= 1)
                    def _():
                        pltpu.make_async_copy(
                            rb.at[1 - p], u_acc.at[pl.ds(0, CH), :],
                            sems_sc[1 - p]).wait()
                    @pl.when(k + 1 < NCH4)
                    def _():
                        pltpu.async_copy(
                            u_out_hbm.at[c].at[edst.at[k + 1, 0]],
                            rb.at[1 - p], sems[1 - p])
                    for jj in range(CH // 16):
                        av = alpha_t[ci, 0, pl.ds(jj * 16, 16)]
                        for i in range(16):
                            r = jj * 16 + i
                            a_i = av[i]
                            for f in range(NGRAPH // 16):
                                rb[p, r, pl.ds(f * 16, 16)] = (
                                    rb[p, r, pl.ds(f * 16, 16)] * a_i)
                    pltpu.async_copy(rb.at[p], u_acc.at[esrc.at[k, 0]],
                                     sems_sc[p], add=True)
                return 0
            lax.fori_loop(0, NCH4 // 2, hop_pair, 0)
            pltpu.make_async_copy(
                rb.at[(NCH4 - 1) % 2], u_acc.at[pl.ds(0, CH), :],
                sems_sc[(NCH4 - 1) % 2]).wait()
            return 0
        lax.fori_loop(0, 4, quarter_body, 0)

        plsc.subcore_barrier()

        def flush_blk(k, _):
            row0 = s * RPT + k * CH
            pltpu.sync_copy(u_acc.at[pl.ds(row0, CH), :], rb.at[0])
            pltpu.sync_copy(rb.at[0], u_out_hbm.at[c, pl.ds(row0, CH), :])
            pltpu.sync_copy(zblk, u_acc.at[pl.ds(row0, CH // 2), :])
            pltpu.sync_copy(zblk, u_acc.at[pl.ds(row0 + CH // 2, CH // 2), :])
            return 0
        lax.fori_loop(0, RPT // CH, flush_blk, 0)
        plsc.subcore_barrier()
        return 0
    lax.fori_loop(0, NHOP, hop_body, 0)


def _sc_propagate(srcp, dstp, coefs, batchp):
    mesh = plsc.VectorSubcoreMesh(core_axis_name="c", subcore_axis_name="s")
    f = pl.kernel(
        _sc_body,
        mesh=mesh,
        compiler_params=pltpu.CompilerParams(
            needs_layout_passes=False, use_tc_tiling_on_sc=False),
        out_type=jax.ShapeDtypeStruct((HEADS, NPAD, NGRAPH), jnp.float32),
        scratch_types=[
            pltpu.VMEM((NCH4, 1, CH), jnp.int32),
            pltpu.VMEM((NCH4, 1, CH), jnp.int32),
            pltpu.VMEM((NCH, 1, CH), jnp.float32),
            pltpu.VMEM((N,), jnp.float32),
            pltpu.VMEM((N,), jnp.float32),
            pltpu.VMEM((NDR, 16), jnp.float32),
            pltpu.VMEM((NDR,), jnp.int32),
            pltpu.VMEM((RPT,), jnp.int32),
            pltpu.VMEM((2, CH, NGRAPH), jnp.float32),
            pltpu.VMEM((CH // 2, NGRAPH), jnp.float32),
            pltpu.VMEM_SHARED((NPAD, NGRAPH), jnp.float32),
            pltpu.VMEM_SHARED((NDR, 16), jnp.float32),
            pltpu.SemaphoreType.DMA,
            pltpu.SemaphoreType.DMA,
            pltpu.SemaphoreType.DMA,
            pltpu.SemaphoreType.DMA,
        ],
    )
    return f(srcp, dstp, coefs, batchp)


def kernel(x, edge_index, batch, W, a_src, a_dst, bias, lin_w, lin_b):
    src, dst = edge_index[0], edge_index[1]
    h, coef = _project(x, W, a_src, a_dst)

    pad = jnp.zeros((NTILE, EPAD - EPT), jnp.int32)
    srcp = jnp.concatenate([src.reshape(NTILE, EPT), pad], axis=1)
    srcp = srcp.reshape(NTILE, NCH, 1, CH)
    dstp = jnp.concatenate([dst.reshape(NTILE, EPT), pad], axis=1)
    dstp = dstp.reshape(NTILE, NCH, 1, CH)
    coefs = jnp.stack([
        jnp.stack([coef[:, 0], coef[:, 2]]),
        jnp.stack([coef[:, 1], coef[:, 3]]),
    ])
    batchp = jnp.concatenate([batch, jnp.zeros((NPAD - N,), jnp.int32)])

    u5 = _sc_propagate(srcp, dstp, coefs, batchp)

    u = jnp.concatenate([u5[0, :N], u5[1, :N]], axis=1)

    u0 = (batch[:, None] == jnp.arange(NGRAPH)[None, :]).astype(jnp.float32)
    cnt = u0.sum(0)
    inv_cnt = (1.0 / jnp.maximum(cnt, 1.0))[:, None]

    return _finalize(u, h, inv_cnt, lin_w, lin_b, bias)


# speedup vs baseline: 2.9555x; 1.7667x over previous
import functools
import math

import jax
import jax.numpy as jnp
from jax import lax
from jax.experimental import pallas as pl
from jax.experimental.pallas import tpu as pltpu
from jax.experimental.pallas import tpu_sc as plsc

N = 10000
E = 320000
D_IN = 128
HID = 256
HEADS = 2
NHOP = 5
NCLS = 10
NGRAPH = 64

NTILE = 16
EPT = E // NTILE
CH = 128
NCH = 160
NCH4 = NCH // 4
EPAD = NCH * CH
NPAD = 10240
RPT = NPAD // NTILE
NDR = NPAD // 16

_ROWS = 1000


def _proj_body(x_ref, W_ref, a_src_ref, a_dst_ref, h_ref, coef_ref):
    h = jnp.dot(x_ref[...], W_ref[...], preferred_element_type=jnp.float32)
    h_ref[...] = h
    hh = h.reshape(_ROWS, HEADS, HID)
    als = jnp.sum(hh * a_src_ref[...][None], axis=-1)
    ald = jnp.sum(hh * a_dst_ref[...][None], axis=-1)
    coef_ref[...] = jnp.concatenate([als, ald], axis=-1)


def _project(x, W, a_src, a_dst):
    grid = N // _ROWS
    return pl.pallas_call(
        _proj_body,
        grid=(grid,),
        in_specs=[
            pl.BlockSpec((_ROWS, D_IN), lambda i: (i, 0)),
            pl.BlockSpec((D_IN, HEADS * HID), lambda i: (0, 0)),
            pl.BlockSpec((HEADS, HID), lambda i: (0, 0)),
            pl.BlockSpec((HEADS, HID), lambda i: (0, 0)),
        ],
        out_specs=[
            pl.BlockSpec((_ROWS, HEADS * HID), lambda i: (i, 0)),
            pl.BlockSpec((_ROWS, 4), lambda i: (i, 0)),
        ],
        out_shape=[
            jax.ShapeDtypeStruct((N, HEADS * HID), jnp.float32),
            jax.ShapeDtypeStruct((N, 4), jnp.float32),
        ],
    )(x, W, a_src, a_dst)


def _final_body(u_ref, h_ref, inv_ref, lw_ref, b_ref, bias_ref, o_ref):
    acc = jnp.zeros((NGRAPH, NCLS), jnp.float32)
    for hd in range(HEADS):
        u = u_ref[:, hd * NGRAPH:(hd + 1) * NGRAPH]
        hmat = h_ref[:, hd * HID:(hd + 1) * HID]
        pooled = jax.lax.dot_general(
            u, hmat, (((0,), (0,)), ((), ())),
            preferred_element_type=jnp.float32)
        acc = acc + jnp.dot(pooled, lw_ref[...],
                            preferred_element_type=jnp.float32)
    acc = acc * inv_ref[...] / HEADS
    bias_term = jnp.dot(bias_ref[...].reshape(1, HID), lw_ref[...],
                        preferred_element_type=jnp.float32)
    o_ref[...] = acc + bias_term + b_ref[...][None, :]


def _finalize(u, h, inv_cnt, lin_w, lin_b, bias):
    return pl.pallas_call(
        _final_body,
        out_shape=jax.ShapeDtypeStruct((NGRAPH, NCLS), jnp.float32),
    )(u, h, inv_cnt, lin_w, lin_b, bias)


def _att_body(src_hbm, dst_hbm, coef_hbm, alpha_hbm,
              esrc, edst, alpha_t, as_t, ad_t, dl, ri, denom_sp, sem):
    c = lax.axis_index("c")
    s = lax.axis_index("s")
    iota = lax.iota(jnp.int32, 16)

    pltpu.sync_copy(coef_hbm.at[c, 0], as_t)
    pltpu.sync_copy(coef_hbm.at[c, 1], ad_t)

    def zdl(i, _):
        dl[i, :] = jnp.zeros((16,), jnp.float32)
        return 0
    lax.fori_loop(0, NDR, zdl, 0)
    def zri(i, _):
        ri[pl.ds(i * 16, 16)] = iota + i * 16
        return 0
    lax.fori_loop(0, NDR // 16, zri, 0)

    pltpu.sync_copy(dl.at[pl.ds(s * (NDR // NTILE), NDR // NTILE), :],
                    denom_sp.at[pl.ds(s * (NDR // NTILE), NDR // NTILE), :])
    plsc.subcore_barrier()

    for q in range(4):
        pltpu.sync_copy(src_hbm.at[s, pl.ds(q * NCH4, NCH4)], esrc)
        pltpu.sync_copy(dst_hbm.at[s, pl.ds(q * NCH4, NCH4)], edst)

        def att_chunk(k, _):
            ci = q * NCH4 + k
            base = ci * CH
            for jj in range(CH // 16):
                ev_idx = base + jj * 16 + iota
                srcv = esrc[k, 0, pl.ds(jj * 16, 16)]
                dstv = edst[k, 0, pl.ds(jj * 16, 16)]
                av = plsc.load_gather(as_t, [srcv])
                dv16 = plsc.load_gather(ad_t, [dstv])
                e = av + dv16
                e = jnp.where(e < 0.0, e * jnp.float32(0.2), e)
                ex = jnp.exp(e)
                ex = jnp.where(ev_idx < EPT, ex, 0.0)
                alpha_t[ci, 0, pl.ds(jj * 16, 16)] = ex
                plsc.addupdate_scatter(dl, [dstv >> 4, dstv & 15], ex)
            return 0
        lax.fori_loop(0, NCH4, att_chunk, 0)

    pltpu.sync_copy(dl, denom_sp.at[ri], add=True)
    plsc.subcore_barrier()
    pltpu.sync_copy(denom_sp, dl)

    for q in range(4):
        pltpu.sync_copy(dst_hbm.at[s, pl.ds(q * NCH4, NCH4)], edst)

        def div_chunk(k, _):
            ci = q * NCH4 + k
            for jj in range(CH // 16):
                dstv = edst[k, 0, pl.ds(jj * 16, 16)]
                dv16 = plsc.load_gather(dl, [dstv >> 4, dstv & 15])
                ex = alpha_t[ci, 0, pl.ds(jj * 16, 16)]
                alpha_t[ci, 0, pl.ds(jj * 16, 16)] = ex / jnp.maximum(
                    dv16, jnp.float32(1e-16))
            return 0
        lax.fori_loop(0, NCH4, div_chunk, 0)

    pltpu.sync_copy(alpha_t, alpha_hbm.at[c, s])


def _sc_attention(srcp, dstp, coefs):
    mesh = plsc.VectorSubcoreMesh(core_axis_name="c", subcore_axis_name="s")
    f = pl.kernel(
        _att_body,
        mesh=mesh,
        compiler_params=pltpu.CompilerParams(
            needs_layout_passes=False, use_tc_tiling_on_sc=False),
        out_type=jax.ShapeDtypeStruct((HEADS, NTILE, NCH, 1, CH),
                                      jnp.float32),
        scratch_types=[
            pltpu.VMEM((NCH4, 1, CH), jnp.int32),
            pltpu.VMEM((NCH4, 1, CH), jnp.int32),
            pltpu.VMEM((NCH, 1, CH), jnp.float32),
            pltpu.VMEM((N,), jnp.float32),
            pltpu.VMEM((N,), jnp.float32),
            pltpu.VMEM((NDR, 16), jnp.float32),
            pltpu.VMEM((NDR,), jnp.int32),
            pltpu.VMEM_SHARED((NDR, 16), jnp.float32),
            pltpu.SemaphoreType.DMA,
        ],
    )
    return f(srcp, dstp, coefs)


def _hop_body(src_hbm, dst_hbm, alpha_hbm, batch_hbm, u_out_hbm,
              esrc, edst, alphaq, bt, rb, zblk,
              u_cur, u_acc, sem, sem2, sem3, sem4):
    c = lax.axis_index("c")
    s = lax.axis_index("s")
    iota = lax.iota(jnp.int32, 16)
    sems = [sem, sem2]
    sems_sc = [sem3, sem4]

    pltpu.sync_copy(batch_hbm.at[pl.ds(s * RPT, RPT)], bt)

    def zr(i, _):
        for f in range(NGRAPH // 16):
            zblk[i, pl.ds(f * 16, 16)] = jnp.zeros((16,), jnp.float32)
        return 0
    lax.fori_loop(0, CH // 2, zr, 0)

    def u0_block(k, _):
        def u0_row16(rr, _):
            bv = bt[pl.ds(k * CH + rr * 16, 16)]
            for i in range(16):
                g = bv[i]
                for f in range(NGRAPH // 16):
                    rb[0, rr * 16 + i, pl.ds(f * 16, 16)] = jnp.where(
                        iota + (f * 16) == g, 1.0, 0.0)
            return 0
        lax.fori_loop(0, CH // 16, u0_row16, 0)
        row0 = s * RPT + k * CH
        pltpu.sync_copy(rb.at[0], u_cur.at[pl.ds(row0, CH), :])
        pltpu.sync_copy(zblk, u_acc.at[pl.ds(row0, CH // 2), :])
        pltpu.sync_copy(zblk, u_acc.at[pl.ds(row0 + CH // 2, CH // 2), :])
        return 0
    lax.fori_loop(0, RPT // CH, u0_block, 0)

    plsc.subcore_barrier()

    def hop_body(hop, _):
        def quarter_body(q, _):
            off = q * NCH4
            pltpu.sync_copy(src_hbm.at[s, pl.ds(off, NCH4)], esrc)
            pltpu.sync_copy(dst_hbm.at[s, pl.ds(off, NCH4)], edst)
            pltpu.sync_copy(alpha_hbm.at[c, s, pl.ds(off, NCH4)], alphaq)

            pltpu.async_copy(u_cur.at[edst.at[0, 0]], rb.at[0], sems[0])

            def hop_pair(kp, _):
                for p in range(2):
                    k = kp * 2 + p
                    pltpu.make_async_copy(
                        u_cur.at[pl.ds(0, CH), :], rb.at[p], sems[p]).wait()
                    @pl.when(k >= 1)
                    def _():
                        pltpu.make_async_copy(
                            rb.at[1 - p], u_acc.at[pl.ds(0, CH), :],
                            sems_sc[1 - p]).wait()
                    @pl.when(k + 1 < NCH4)
                    def _():
                        pltpu.async_copy(
                            u_cur.at[edst.at[k + 1, 0]], rb.at[1 - p],
                            sems[1 - p])
                    for jj in range(CH // 16):
                        av = alphaq[k, 0, pl.ds(jj * 16, 16)]
                        for i in range(16):
                            r = jj * 16 + i
                            a_i = av[i]
                            for f in range(NGRAPH // 16):
                                rb[p, r, pl.ds(f * 16, 16)] = (
                                    rb[p, r, pl.ds(f * 16, 16)] * a_i)
                    pltpu.async_copy(rb.at[p], u_acc.at[esrc.at[k, 0]],
                                     sems_sc[p], add=True)
                return 0
            lax.fori_loop(0, NCH4 // 2, hop_pair, 0)
            pltpu.make_async_copy(
                rb.at[(NCH4 - 1) % 2], u_acc.at[pl.ds(0, CH), :],
                sems_sc[(NCH4 - 1) % 2]).wait()
            return 0
        lax.fori_loop(0, 4, quarter_body, 0)

        plsc.subcore_barrier()

        def flush_blk(k, _):
            row0 = s * RPT + k * CH
            pltpu.sync_copy(u_acc.at[pl.ds(row0, CH), :], rb.at[0])
            @pl.when(hop == NHOP - 1)
            def _():
                pltpu.sync_copy(rb.at[0],
                                u_out_hbm.at[c, pl.ds(row0, CH), :])
            @pl.when(hop < NHOP - 1)
            def _():
                pltpu.sync_copy(rb.at[0], u_cur.at[pl.ds(row0, CH), :])
                pltpu.sync_copy(zblk, u_acc.at[pl.ds(row0, CH // 2), :])
                pltpu.sync_copy(
                    zblk, u_acc.at[pl.ds(row0 + CH // 2, CH // 2), :])
            return 0
        lax.fori_loop(0, RPT // CH, flush_blk, 0)
        plsc.subcore_barrier()
        return 0
    lax.fori_loop(0, NHOP, hop_body, 0)


def _sc_hops(srcp, dstp, alpha, batchp):
    mesh = plsc.VectorSubcoreMesh(core_axis_name="c", subcore_axis_name="s")
    f = pl.kernel(
        _hop_body,
        mesh=mesh,
        compiler_params=pltpu.CompilerParams(
            needs_layout_passes=False, use_tc_tiling_on_sc=False),
        out_type=jax.ShapeDtypeStruct((HEADS, NPAD, NGRAPH), jnp.float32),
        scratch_types=[
            pltpu.VMEM((NCH4, 1, CH), jnp.int32),
            pltpu.VMEM((NCH4, 1, CH), jnp.int32),
            pltpu.VMEM((NCH4, 1, CH), jnp.float32),
            pltpu.VMEM((RPT,), jnp.int32),
            pltpu.VMEM((2, CH, NGRAPH), jnp.float32),
            pltpu.VMEM((CH // 2, NGRAPH), jnp.float32),
            pltpu.VMEM_SHARED((NPAD, NGRAPH), jnp.float32),
            pltpu.VMEM_SHARED((NPAD, NGRAPH), jnp.float32),
            pltpu.SemaphoreType.DMA,
            pltpu.SemaphoreType.DMA,
            pltpu.SemaphoreType.DMA,
            pltpu.SemaphoreType.DMA,
        ],
    )
    return f(srcp, dstp, alpha, batchp)


def kernel(x, edge_index, batch, W, a_src, a_dst, bias, lin_w, lin_b):
    src, dst = edge_index[0], edge_index[1]
    h, coef = _project(x, W, a_src, a_dst)

    pad = jnp.zeros((NTILE, EPAD - EPT), jnp.int32)
    srcp = jnp.concatenate([src.reshape(NTILE, EPT), pad], axis=1)
    srcp = srcp.reshape(NTILE, NCH, 1, CH)
    dstp = jnp.concatenate([dst.reshape(NTILE, EPT), pad], axis=1)
    dstp = dstp.reshape(NTILE, NCH, 1, CH)
    coefs = jnp.stack([
        jnp.stack([coef[:, 0], coef[:, 2]]),
        jnp.stack([coef[:, 1], coef[:, 3]]),
    ])
    batchp = jnp.concatenate([batch, jnp.zeros((NPAD - N,), jnp.int32)])

    alpha = _sc_attention(srcp, dstp, coefs)
    u5 = _sc_hops(srcp, dstp, alpha, batchp)

    u = jnp.concatenate([u5[0, :N], u5[1, :N]], axis=1)

    u0 = (batch[:, None] == jnp.arange(NGRAPH)[None, :]).astype(jnp.float32)
    cnt = u0.sum(0)
    inv_cnt = (1.0 / jnp.maximum(cnt, 1.0))[:, None]

    return _finalize(u, h, inv_cnt, lin_w, lin_b, bias)
